# baseline (speedup 1.0000x reference)
import functools

import jax
import jax.numpy as jnp
from jax import lax
from jax.experimental import pallas as pl
from jax.experimental.pallas import tpu as pltpu
from jax.experimental.pallas import tpu_sc as plsc

_D = 64
_NC = 2
_NS = 16
_NW = _NC * _NS
_S = 50
_B = 4096
_L = 128
_DT = _D // 8


@functools.partial(jax.jit, static_argnames=("total",))
def _gather(weight, idx, total):
    del total
    mesh = plsc.VectorSubcoreMesh(core_axis_name="c", subcore_axis_name="s")
    b_per_w = _L * _S

    @functools.partial(
        pl.kernel,
        mesh=mesh,
        out_type=jax.ShapeDtypeStruct((_S, _DT, _B // _L, 8 * _L),
                                      jnp.float32),
        scratch_types=[
            pltpu.VMEM((b_per_w,), jnp.int32),
            pltpu.VMEM((_S, _L), jnp.int32),
            pltpu.VMEM((4, _L, _D), jnp.float32),
            pltpu.VMEM((4, _D * _L), jnp.float32),
            pltpu.SemaphoreType.DMA,
            pltpu.SemaphoreType.DMA,
            pltpu.SemaphoreType.DMA,
            pltpu.SemaphoreType.DMA,
            pltpu.SemaphoreType.DMA,
            pltpu.SemaphoreType.DMA,
            pltpu.SemaphoreType.DMA,
            pltpu.SemaphoreType.DMA,
        ],
        compiler_params=pltpu.CompilerParams(
            use_tc_tiling_on_sc=False, needs_layout_passes=False),
    )
    def gather_kernel(table_hbm, idx_hbm, out5_hbm, idx_v, idxt_v, rows_v,
                      tile_v, sem_g0, sem_g1, sem_g2, sem_g3,
                      sem_o0, sem_o1, sem_o2, sem_o3):
        sem_g = [sem_g0, sem_g1, sem_g2, sem_g3]
        sem_o = [sem_o0, sem_o1, sem_o2, sem_o3]
        bt = lax.axis_index("s") * _NC + lax.axis_index("c")
        iota = lax.iota(jnp.int32, 16)

        pltpu.sync_copy(idx_hbm.at[pl.ds(bt * b_per_w, b_per_w)], idx_v)

        def tr_idx(l0, carry):
            src = iota * _S + l0 * 16 * _S
            for s in range(_S):
                idxt_v[s, pl.ds(l0 * 16, 16)] = plsc.load_gather(
                    idx_v, [src + s])
            return carry

        lax.fori_loop(0, _L // 16, tr_idx, 0)

        def gather_desc(s, buf, sem):
            return pltpu.make_async_copy(
                table_hbm.at[idxt_v.at[s]], rows_v.at[buf], sem)

        def write_descs(s, buf, sem):
            return [
                pltpu.make_async_copy(
                    tile_v.at[buf, pl.ds(dt * 8 * _L, 8 * _L)],
                    out5_hbm.at[s, dt, bt], sem)
                for dt in range(_DT)
            ]

        _mk = [(iota + k) % 16 for k in range(16)]
        _sk = [m * _L + iota for m in _mk]

        def transpose_rows(buf):
            rows = rows_v.at[buf]
            tile = tile_v.at[buf]

            @plsc.parallel_loop(0, (_L // 16) * (_D // 16), unroll=4)
            def tr(b0):
                r0 = (b0 // (_D // 16)) * 16
                d0 = (b0 % (_D // 16)) * 16
                rvec = r0 + iota
                sbase = d0 * _L + r0
                for k in range(16):
                    v = plsc.load_gather(rows, [rvec, _mk[k] + d0])
                    plsc.store_scatter(tile, [_sk[k] + sbase], v)

        n_buf = 4
        n_k = 12

        for s0 in range(n_buf):
            gather_desc(s0, s0, sem_g[s0]).start()

        def quarter(k, s, q):
            gather_desc(s, q, sem_g[q]).wait()

            @pl.when(k > 0)
            def _():
                for d in write_descs(s, q, sem_o[q]):
                    d.wait()

            transpose_rows(q)
            for d in write_descs(s, q, sem_o[q]):
                d.start()
            if q < 2:
                gather_desc(s + n_buf, q, sem_g[q]).start()
            else:
                @pl.when(k < n_k - 1)
                def _():
                    gather_desc(s + n_buf, q, sem_g[q]).start()

        def body(k, carry):
            for q in range(n_buf):
                quarter(k, n_buf * k + q, q)
            return carry

        lax.fori_loop(0, n_k, body, 0)

        for s, q in ((_S - 2, 0), (_S - 1, 1)):
            gather_desc(s, q, sem_g[q]).wait()
            for d in write_descs(s, q, sem_o[q]):
                d.wait()
            transpose_rows(q)
            for d in write_descs(s, q, sem_o[q]):
                d.start()

        for s, q in ((_S - 4, 2), (_S - 3, 3), (_S - 2, 0), (_S - 1, 1)):
            for d in write_descs(s, q, sem_o[q]):
                d.wait()

    return gather_kernel(weight, idx)


def kernel(token_ids, weight):
    idx = token_ids.reshape(-1).astype(jnp.int32)
    out4 = _gather(weight, idx, idx.shape[0])
    out5 = out4.reshape(_S, _DT, _B // _L, 8, _L)
    return out5.transpose(2, 4, 0, 1, 3).reshape(_B, _S, _D)


# speedup vs baseline: 1.1999x; 1.1999x over previous
import functools

import jax
import jax.numpy as jnp
from jax import lax
from jax.experimental import pallas as pl
from jax.experimental.pallas import tpu as pltpu
from jax.experimental.pallas import tpu_sc as plsc

_D = 64
_NC = 2
_NS = 16
_NW = _NC * _NS
_S = 50
_B = 4096
_L = 128
_DT = _D // 8


@functools.partial(jax.jit, static_argnames=("total",))
def _gather(weight, idx, total):
    del total
    mesh = plsc.VectorSubcoreMesh(core_axis_name="c", subcore_axis_name="s")
    b_per_w = _L * _S

    @functools.partial(
        pl.kernel,
        mesh=mesh,
        out_type=jax.ShapeDtypeStruct((_S, _DT, _B // _L, 8 * _L),
                                      jnp.float32),
        scratch_types=[
            pltpu.VMEM((b_per_w,), jnp.int32),
            pltpu.VMEM((_S, _L), jnp.int32),
            pltpu.VMEM((4, _L, _D), jnp.float32),
            pltpu.VMEM((4, _D * _L), jnp.float32),
            pltpu.SemaphoreType.DMA,
            pltpu.SemaphoreType.DMA,
            pltpu.SemaphoreType.DMA,
            pltpu.SemaphoreType.DMA,
            pltpu.SemaphoreType.DMA,
            pltpu.SemaphoreType.DMA,
            pltpu.SemaphoreType.DMA,
            pltpu.SemaphoreType.DMA,
        ],
        compiler_params=pltpu.CompilerParams(
            use_tc_tiling_on_sc=False, needs_layout_passes=False),
    )
    def gather_kernel(table_hbm, idx_hbm, out5_hbm, idx_v, idxt_v, rows_v,
                      tile_v, sem_g0, sem_g1, sem_g2, sem_g3,
                      sem_o0, sem_o1, sem_o2, sem_o3):
        sem_g = [sem_g0, sem_g1, sem_g2, sem_g3]
        sem_o = [sem_o0, sem_o1, sem_o2, sem_o3]
        bt = lax.axis_index("s") * _NC + lax.axis_index("c")
        iota = lax.iota(jnp.int32, 16)

        pltpu.sync_copy(idx_hbm.at[pl.ds(bt * b_per_w, b_per_w)], idx_v)

        def tr_idx(l0, carry):
            src = iota * _S + l0 * 16 * _S
            for s in range(_S):
                idxt_v[s, pl.ds(l0 * 16, 16)] = plsc.load_gather(
                    idx_v, [src + s])
            return carry

        lax.fori_loop(0, _L // 16, tr_idx, 0)

        def gather_desc(s, buf, sem):
            return pltpu.make_async_copy(
                table_hbm.at[idxt_v.at[s]], rows_v.at[buf], sem)

        def write_descs(s, buf, sem):
            return [
                pltpu.make_async_copy(
                    tile_v.at[buf, pl.ds(dt * 8 * _L, 8 * _L)],
                    out5_hbm.at[s, dt, bt], sem)
                for dt in range(_DT)
            ]

        _mk = [(iota + k) % 16 for k in range(16)]
        _sk = [m * _L + iota for m in _mk]

        def transpose_rows(buf):
            rows = rows_v.at[buf]
            tile = tile_v.at[buf]

            @plsc.parallel_loop(0, (_L // 16) * (_D // 16), unroll=2)
            def tr(b0):
                r0 = (b0 // (_D // 16)) * 16
                d0 = (b0 % (_D // 16)) * 16
                rvec = r0 + iota
                sbase = d0 * _L + r0
                for k in range(16):
                    v = plsc.load_gather(rows, [rvec, _mk[k] + d0])
                    plsc.store_scatter(tile, [_sk[k] + sbase], v)

        n_buf = 4
        n_k = 12

        for s0 in range(n_buf):
            gather_desc(s0, s0, sem_g[s0]).start()

        def quarter(k, s, q):
            gather_desc(s, q, sem_g[q]).wait()

            @pl.when(k > 0)
            def _():
                for d in write_descs(s, q, sem_o[q]):
                    d.wait()

            transpose_rows(q)
            for d in write_descs(s, q, sem_o[q]):
                d.start()
            if q < 2:
                gather_desc(s + n_buf, q, sem_g[q]).start()
            else:
                @pl.when(k < n_k - 1)
                def _():
                    gather_desc(s + n_buf, q, sem_g[q]).start()

        def body(k, carry):
            for q in range(n_buf):
                quarter(k, n_buf * k + q, q)
            return carry

        lax.fori_loop(0, n_k, body, 0)

        for s, q in ((_S - 2, 0), (_S - 1, 1)):
            gather_desc(s, q, sem_g[q]).wait()
            for d in write_descs(s, q, sem_o[q]):
                d.wait()
            transpose_rows(q)
            for d in write_descs(s, q, sem_o[q]):
                d.start()

        for s, q in ((_S - 4, 2), (_S - 3, 3), (_S - 2, 0), (_S - 1, 1)):
            for d in write_descs(s, q, sem_o[q]):
                d.wait()

    return gather_kernel(weight, idx)


def kernel(token_ids, weight):
    idx = token_ids.reshape(-1).astype(jnp.int32)
    out4 = _gather(weight, idx, idx.shape[0])
    out5 = out4.reshape(_S, _DT, _B // _L, 8, _L)
    return out5.transpose(2, 4, 0, 1, 3).reshape(_B, _S, _D)
